# Initial kernel scaffold; baseline (speedup 1.0000x reference)
import jax
import jax.numpy as jnp
from jax.experimental import pallas as pl


def kernel(hidden_states, gate_proj, up_proj, down_proj, token_ids, token_to_expert):
    raise NotImplementedError("write your pallas kernel here")


# speedup vs baseline: 1.6461x; 1.6461x over previous
import functools

import jax
import jax.numpy as jnp
from jax.experimental import pallas as pl
from jax.experimental.pallas import tpu as pltpu

TM = 256
NE = 8
NT_PAD = 0


def _mlp1_body(te_ref, x_ref, wg_ref, wu_ref, o_ref):
    x = x_ref[...]
    g = jax.lax.dot_general(x, wg_ref[0], (((1,), (0,)), ((), ())),
                            preferred_element_type=jnp.float32)
    u = jax.lax.dot_general(x, wu_ref[0], (((1,), (0,)), ((), ())),
                            preferred_element_type=jnp.float32)
    o_ref[...] = (g * jax.lax.logistic(g)) * u


def _mlp2_body(te_ref, h_ref, wd_ref, o_ref):
    o_ref[...] = jax.lax.dot_general(h_ref[...], wd_ref[0],
                                     (((1,), (0,)), ((), ())),
                                     preferred_element_type=jnp.float32)


def kernel(hidden_states, gate_proj, up_proj, down_proj, token_ids, token_to_expert):
    Bb, Ss, H = hidden_states.shape
    nE, _, EI = gate_proj.shape
    V = token_to_expert.shape[0]
    T = Bb * Ss
    P = T + nE * TM
    NT = P // TM

    flat_x = hidden_states.reshape(T, H)
    ids = jnp.clip(token_ids.reshape(-1), 0, V - 1)
    eids = jnp.take(token_to_expert, ids, axis=0)

    order = jnp.argsort(eids, stable=True)
    sorted_e = eids[order]
    counts = jnp.bincount(eids, length=nE)
    padded = ((counts + TM - 1) // TM) * TM
    start = jnp.concatenate([jnp.zeros(1, jnp.int32),
                             jnp.cumsum(padded)[:-1].astype(jnp.int32)])
    cstart = jnp.concatenate([jnp.zeros(1, jnp.int32),
                              jnp.cumsum(counts)[:-1].astype(jnp.int32)])
    pos = start[sorted_e] + (jnp.arange(T, dtype=jnp.int32) - cstart[sorted_e])
    tile_ids = jnp.arange(NT, dtype=jnp.int32) * TM
    ends = jnp.cumsum(padded).astype(jnp.int32)
    tile_expert = jnp.minimum(jnp.searchsorted(ends, tile_ids, side="right"),
                              nE - 1).astype(jnp.int32)
    total_padded = ends[-1]
    first_e = sorted_e[0]
    tile_expert = jnp.where(tile_ids < total_padded, tile_expert, first_e)
    clone_tok = order[jnp.clip(cstart[tile_expert], 0, T - 1)]
    perm = jnp.repeat(clone_tok, TM, total_repeat_length=P)
    perm = perm.at[pos].set(order)

    x_sorted = jnp.take(flat_x, perm, axis=0)

    grid1 = pltpu.PrefetchScalarGridSpec(
        num_scalar_prefetch=1,
        grid=(NT,),
        in_specs=[
            pl.BlockSpec((TM, H), lambda i, te: (i, 0)),
            pl.BlockSpec((1, H, EI), lambda i, te: (te[i], 0, 0)),
            pl.BlockSpec((1, H, EI), lambda i, te: (te[i], 0, 0)),
        ],
        out_specs=pl.BlockSpec((TM, EI), lambda i, te: (i, 0)),
    )
    inter = pl.pallas_call(
        _mlp1_body, grid_spec=grid1,
        out_shape=jax.ShapeDtypeStruct((P, EI), jnp.float32),
    )(tile_expert, x_sorted, gate_proj, up_proj)

    grid2 = pltpu.PrefetchScalarGridSpec(
        num_scalar_prefetch=1,
        grid=(NT,),
        in_specs=[
            pl.BlockSpec((TM, EI), lambda i, te: (i, 0)),
            pl.BlockSpec((1, EI, H), lambda i, te: (te[i], 0, 0)),
        ],
        out_specs=pl.BlockSpec((TM, H), lambda i, te: (i, 0)),
    )
    y_sorted = pl.pallas_call(
        _mlp2_body, grid_spec=grid2,
        out_shape=jax.ShapeDtypeStruct((P, H), jnp.float32),
    )(tile_expert, inter, down_proj)

    out = jnp.zeros((T, H), jnp.float32).at[perm].set(y_sorted)
    return out.reshape(Bb, Ss, H)
